# Initial kernel scaffold; baseline (speedup 1.0000x reference)
import jax
import jax.numpy as jnp
from jax.experimental import pallas as pl


def kernel(x, W_down, W_up, input_gate):
    raise NotImplementedError("write your pallas kernel here")


# speedup vs baseline: 1.3008x; 1.3008x over previous
import jax
import jax.numpy as jnp
from jax.experimental import pallas as pl
from jax.experimental.pallas import tpu as pltpu

LORA_ALPHA = 128.0


def _body(x_ref, g_ref, wd_ref, wu_ref, o_ref, *, scale):
    xb = x_ref[...]
    gs = jnp.sum(xb * g_ref[...], axis=-1, keepdims=True)
    h = jnp.dot(xb, wd_ref[...], preferred_element_type=jnp.float32)
    h = h * (jax.nn.sigmoid(gs) * scale)
    o_ref[...] = jnp.dot(h, wu_ref[...], preferred_element_type=jnp.float32)


def kernel(x, W_down, W_up, input_gate):
    B, S, D_IN = x.shape
    R = W_down.shape[0]
    D_OUT = W_up.shape[0]
    scale = LORA_ALPHA / R

    T = B * S
    TILE = 1024
    xf = x.reshape(T, D_IN)
    wd = W_down.T
    wu = W_up.T
    g = input_gate.reshape(1, D_IN)

    out = pl.pallas_call(
        lambda *refs: _body(*refs, scale=scale),
        grid=(T // TILE,),
        in_specs=[
            pl.BlockSpec((TILE, D_IN), lambda i: (i, 0)),
            pl.BlockSpec((1, D_IN), lambda i: (0, 0)),
            pl.BlockSpec((D_IN, R), lambda i: (0, 0)),
            pl.BlockSpec((R, D_OUT), lambda i: (0, 0)),
        ],
        out_specs=pl.BlockSpec((TILE, D_OUT), lambda i: (i, 0)),
        out_shape=jax.ShapeDtypeStruct((T, D_OUT), jnp.float32),
        compiler_params=pltpu.CompilerParams(
            dimension_semantics=("parallel",),
        ),
    )(xf, g, wd, wu)

    return out.reshape(B, S, D_OUT)
